# baseline (speedup 1.0000x reference)
import jax
import jax.numpy as jnp
from jax.experimental import pallas as pl
from jax.experimental.pallas import tpu as pltpu


def _copy_body(in_ref, out_ref):
    out_ref[...] = in_ref[...]


def _copy3d(x, block):
    n, s, width = x.shape
    grid = n // block
    return pl.pallas_call(
        _copy_body,
        out_shape=jax.ShapeDtypeStruct(x.shape, x.dtype),
        grid=(grid,),
        in_specs=[pl.BlockSpec((block, s, width), lambda i: (i, 0, 0))],
        out_specs=pl.BlockSpec((block, s, width), lambda i: (i, 0, 0)),
    )(x)


def kernel(emb_user, emb_item):
    u_shape, i_shape = emb_user.shape, emb_item.shape
    u3 = emb_user.reshape(-1, 8, 128)
    i3 = emb_item.reshape(-1, 8, 128)
    out_u = _copy3d(u3, 2500)
    out_i = _copy3d(i3, 1250)
    return (out_u.reshape(u_shape), out_i.reshape(i_shape))


# speedup vs baseline: 1.2360x; 1.2360x over previous
import jax
import jax.numpy as jnp
from jax import lax
from jax.experimental import pallas as pl
from jax.experimental.pallas import tpu as pltpu
from jax.experimental.pallas import tpu_sc as plsc

_B = 400
_NW = 32


def _sc_copy_body(u_in, i_in, u_out, i_out, bufs, in_sems, out_sems):
    wid = lax.axis_index("c") * 16 + lax.axis_index("s")

    def phase(src, dst, n_chunks):
        iters = (n_chunks + _NW - 1) // _NW

        def masked(j, fn):
            c = wid + _NW * j

            @pl.when(c < n_chunks)
            def _():
                fn(c)

        def in_copy(j, c):
            return pltpu.make_async_copy(
                src.at[pl.ds(c * _B, _B)], bufs.at[j % 2], in_sems.at[j % 2])

        def out_copy(j, c):
            return pltpu.make_async_copy(
                bufs.at[j % 2], dst.at[pl.ds(c * _B, _B)], out_sems.at[j % 2])

        masked(0, lambda c: in_copy(0, c).start())
        for j in range(iters):
            masked(j, lambda c, j=j: in_copy(j, c).wait())
            masked(j, lambda c, j=j: out_copy(j, c).start())
            if j + 1 < iters:
                if j >= 1:
                    masked(j - 1, lambda c, j=j: out_copy(j - 1, c).wait())
                masked(j + 1, lambda c, j=j: in_copy(j + 1, c).start())
        if iters >= 2:
            masked(iters - 2, lambda c: out_copy(iters - 2, c).wait())
        if iters:
            masked(iters - 1, lambda c: out_copy(iters - 1, c).wait())

    phase(u_in, u_out, u_in.shape[0] // _B)
    phase(i_in, i_out, i_in.shape[0] // _B)


def kernel(emb_user, emb_item):
    mesh = plsc.VectorSubcoreMesh(core_axis_name="c", subcore_axis_name="s")
    run = pl.kernel(
        _sc_copy_body,
        out_type=(
            jax.ShapeDtypeStruct(emb_user.shape, emb_user.dtype),
            jax.ShapeDtypeStruct(emb_item.shape, emb_item.dtype),
        ),
        mesh=mesh,
        scratch_types=[
            pltpu.VMEM((2, _B, 64), jnp.float32),
            pltpu.SemaphoreType.DMA((2,)),
            pltpu.SemaphoreType.DMA((2,)),
        ],
        compiler_params=pltpu.CompilerParams(use_tc_tiling_on_sc=True),
    )
    return run(emb_user, emb_item)


# speedup vs baseline: 1.7496x; 1.4156x over previous
import jax
import jax.numpy as jnp
from jax.experimental import pallas as pl
from jax.experimental.pallas import tpu as pltpu

_T = 32


def _copy_body(in_ref, out_ref):
    out_ref[...] = in_ref[...]


def _copy3d(x, block):
    n, t, width = x.shape
    grid = n // block
    return pl.pallas_call(
        _copy_body,
        out_shape=jax.ShapeDtypeStruct(x.shape, x.dtype),
        grid=(grid,),
        in_specs=[pl.BlockSpec((block, t, width), lambda i: (i, 0, 0))],
        out_specs=pl.BlockSpec((block, t, width), lambda i: (i, 0, 0)),
        compiler_params=pltpu.CompilerParams(
            vmem_limit_bytes=60 * 1024 * 1024,
        ),
    )(x)


def kernel(emb_user, emb_item):
    u_shape, i_shape = emb_user.shape, emb_item.shape
    u3 = emb_user.reshape(-1, _T, 64)
    i3 = emb_item.reshape(-1, _T, 64)
    out_u = _copy3d(u3, 625)
    out_i = _copy3d(i3, 625)
    return (out_u.reshape(u_shape), out_i.reshape(i_shape))
